# baseline (speedup 1.0000x reference)
import jax
import jax.numpy as jnp
from jax import lax
from jax.experimental import pallas as pl
from jax.experimental.pallas import tpu as pltpu

_B = 128
_N = 4096
_K = 8
_NBLK = 8
_BO = _N // _NBLK

_NEG = float("-inf")


def _mm_body(field_ref, w_ref, b_ref, out_ref):
    blk = lax.dot_general(
        field_ref[...], w_ref[...], (((1,), (1,)), ((), ())),
        preferred_element_type=jnp.float32,
    )
    out_ref[...] = blk + b_ref[...].reshape(1, _BO)


def _sel_body(field_ref, s_ref, out_ref):
    f = field_ref[...]
    s = s_ref[...]
    col = lax.broadcasted_iota(jnp.int32, (_B, _N), 1)
    for k in range(_K):
        idx = jnp.argmax(s, axis=1)[:, None]
        hit = col == idx
        out_ref[:, k] = jnp.max(jnp.where(hit, f, _NEG), axis=1)
        if k < _K - 1:
            s = jnp.where(hit, _NEG, s)


def kernel(field, W, b, training):
    del training
    scores = pl.pallas_call(
        _mm_body,
        grid=(_NBLK,),
        in_specs=[
            pl.BlockSpec((_B, _N), lambda i: (0, 0)),
            pl.BlockSpec((_BO, _N), lambda i: (i, 0)),
            pl.BlockSpec((_BO,), lambda i: (i,)),
        ],
        out_specs=pl.BlockSpec((_B, _BO), lambda i: (0, i)),
        out_shape=jax.ShapeDtypeStruct((_B, _N), jnp.float32),
        compiler_params=pltpu.CompilerParams(
            dimension_semantics=("arbitrary",),
        ),
    )(field, W, b)
    return pl.pallas_call(
        _sel_body,
        out_shape=jax.ShapeDtypeStruct((_B, _K), jnp.float32),
    )(field, scores)


# speedup vs baseline: 1.0983x; 1.0983x over previous
import jax
import jax.numpy as jnp
from jax import lax
from jax.experimental import pallas as pl
from jax.experimental.pallas import tpu as pltpu

_B = 128
_N = 4096
_K = 8
_NBLK = 8
_BO = _N // _NBLK

_NEG = float("-inf")


def _body(field_ref, w_ref, b_ref, out_ref, scores_ref):
    i = pl.program_id(0)
    f = field_ref[...]
    wblk = w_ref[...]
    blk = lax.dot_general(
        f, wblk, (((1,), (1,)), ((), ())), preferred_element_type=jnp.float32
    )
    scores_ref[:, pl.ds(i * _BO, _BO)] = blk + b_ref[...].reshape(1, _BO)

    @pl.when(i == _NBLK - 1)
    def _select():
        s = scores_ref[...]
        col = lax.broadcasted_iota(jnp.int32, (_B, _N), 1)
        for k in range(_K):
            idx = jnp.argmax(s, axis=1)[:, None]
            hit = col == idx
            out_ref[:, k] = jnp.max(jnp.where(hit, f, _NEG), axis=1)
            if k < _K - 1:
                s = jnp.where(hit, _NEG, s)


def kernel(field, W, b, training):
    del training
    return pl.pallas_call(
        _body,
        grid=(_NBLK,),
        in_specs=[
            pl.BlockSpec((_B, _N), lambda i: (0, 0)),
            pl.BlockSpec((_BO, _N), lambda i: (i, 0)),
            pl.BlockSpec((_BO,), lambda i: (i,)),
        ],
        out_specs=pl.BlockSpec((_B, _K), lambda i: (0, 0)),
        out_shape=jax.ShapeDtypeStruct((_B, _K), jnp.float32),
        scratch_shapes=[pltpu.VMEM((_B, _N), jnp.float32)],
        compiler_params=pltpu.CompilerParams(
            dimension_semantics=("arbitrary",),
        ),
    )(field, W, b)
